# baseline (speedup 1.0000x reference)
import functools

import jax
import jax.numpy as jnp
from jax import lax
from jax.experimental import pallas as pl
from jax.experimental.pallas import tpu as pltpu
from jax.experimental.pallas import tpu_sc as plsc

L, T, D = 32, 1024, 1024
LANES = 16
NCHUNK = T // LANES


@functools.partial(
    pl.kernel,
    out_type=jax.ShapeDtypeStruct((LANES,), jnp.int32),
    mesh=plsc.VectorSubcoreMesh(core_axis_name="c", subcore_axis_name="s",
                                num_cores=2, num_subcores=16),
    scratch_types=[
        pltpu.VMEM((LANES,), jnp.float32),
        pltpu.VMEM((T,), jnp.int32),
        pltpu.VMEM((LANES,), jnp.int32),
    ],
)
def _sc_slot_scan(layer_hbm, bstep_hbm, slot_hbm, lay_v, row_v, out_v):
    cid = lax.axis_index("c")
    sid = lax.axis_index("s")
    is_owner = (cid == 0) & (sid == 0)

    @pl.when(is_owner)
    def _():
        pltpu.sync_copy(layer_hbm, lay_v)
        layer = lay_v[...][0].astype(jnp.int32)
        pltpu.sync_copy(bstep_hbm.at[layer], row_v)

        iota_f = lax.iota(jnp.int32, LANES).astype(jnp.float32)
        bigf = jnp.float32(1e9)

        def body(i, carry):
            acc_occ, acc_emp = carry
            v = row_v[pl.ds(i * LANES, LANES)]
            gidx_f = iota_f + jnp.float32(i * LANES)
            enc = v.astype(jnp.float32) * jnp.float32(T) + gidx_f
            acc_occ = jnp.minimum(acc_occ, enc)
            acc_emp = jnp.minimum(acc_emp, jnp.where(v == -1, gidx_f, bigf))
            return acc_occ, acc_emp

        acc_occ, acc_emp = lax.fori_loop(
            0, NCHUNK, body,
            (jnp.full((LANES,), 1e9, jnp.float32),
             jnp.full((LANES,), 1e9, jnp.float32)))
        m_occ = acc_occ[0]
        m_emp = acc_emp[0]
        for j in range(1, LANES):
            m_occ = jnp.minimum(m_occ, acc_occ[j])
            m_emp = jnp.minimum(m_emp, acc_emp[j])
        slot_occ = m_occ.astype(jnp.int32) & (T - 1)
        slot = jnp.where(m_emp < bigf, m_emp.astype(jnp.int32), slot_occ)

        out_v[...] = jnp.full((LANES,), 0, jnp.int32) + slot
        pltpu.sync_copy(out_v, slot_hbm)


def _scatter_kernel(layer_ref, step_ref, ec_ref, slot_ref, ev_ref, bev_in_ref,
                    bstep_ref, bec_ref, bev_out_ref, bstep_out_ref,
                    bec_out_ref, sem):
    del bev_in_ref
    layer = layer_ref[0]
    step = step_ref[0]
    ec = ec_ref[0]
    slot = slot_ref[0]

    row_iota = jax.lax.broadcasted_iota(jnp.int32, (L, T), 0)
    col_iota = jax.lax.broadcasted_iota(jnp.int32, (L, T), 1)
    hit = (row_iota == layer) & (col_iota == slot)
    bstep_out_ref[...] = jnp.where(hit, step, bstep_ref[...])
    bec_out_ref[...] = jnp.where(hit, ec, bec_ref[...])

    copy = pltpu.make_async_copy(ev_ref.at[0], bev_out_ref.at[layer, slot], sem)
    copy.start()
    copy.wait()


def kernel(layer, step, evidence, event_count, bank_evidence, bank_step,
           bank_event_count):
    layer_s = jnp.asarray(layer, jnp.int32).reshape(1)
    step_s = jnp.asarray(step, bank_step.dtype).reshape(1)
    ec_s = jnp.asarray(event_count, bank_event_count.dtype).reshape(1)
    lay16 = jnp.full((LANES,), layer, jnp.float32)
    ev2 = evidence.astype(bank_evidence.dtype).reshape(1, D)

    slot16 = _sc_slot_scan(lay16, bank_step)

    return pl.pallas_call(
        _scatter_kernel,
        out_shape=(
            jax.ShapeDtypeStruct(bank_evidence.shape, bank_evidence.dtype),
            jax.ShapeDtypeStruct(bank_step.shape, bank_step.dtype),
            jax.ShapeDtypeStruct(bank_event_count.shape, bank_event_count.dtype),
        ),
        in_specs=[
            pl.BlockSpec(memory_space=pltpu.MemorySpace.SMEM),
            pl.BlockSpec(memory_space=pltpu.MemorySpace.SMEM),
            pl.BlockSpec(memory_space=pltpu.MemorySpace.SMEM),
            pl.BlockSpec(memory_space=pltpu.MemorySpace.SMEM),
            pl.BlockSpec(memory_space=pltpu.MemorySpace.VMEM),
            pl.BlockSpec(memory_space=pltpu.MemorySpace.HBM),
            pl.BlockSpec(memory_space=pltpu.MemorySpace.VMEM),
            pl.BlockSpec(memory_space=pltpu.MemorySpace.VMEM),
        ],
        out_specs=(
            pl.BlockSpec(memory_space=pltpu.MemorySpace.HBM),
            pl.BlockSpec(memory_space=pltpu.MemorySpace.VMEM),
            pl.BlockSpec(memory_space=pltpu.MemorySpace.VMEM),
        ),
        input_output_aliases={5: 0},
        scratch_shapes=[pltpu.SemaphoreType.DMA],
    )(layer_s, step_s, ec_s, slot16[:1], ev2, bank_evidence, bank_step,
      bank_event_count)


# speedup vs baseline: 1.0076x; 1.0076x over previous
import functools

import jax
import jax.numpy as jnp
from jax import lax
from jax.experimental import pallas as pl
from jax.experimental.pallas import tpu as pltpu
from jax.experimental.pallas import tpu_sc as plsc

L, T, D = 32, 1024, 1024
LANES = 16
NCHUNK = T // LANES


@functools.partial(
    pl.kernel,
    out_type=jax.ShapeDtypeStruct((L, LANES), jnp.int32),
    mesh=plsc.VectorSubcoreMesh(core_axis_name="c", subcore_axis_name="s",
                                num_cores=2, num_subcores=16),
    scratch_types=[
        pltpu.VMEM((T,), jnp.int32),
        pltpu.VMEM((LANES,), jnp.int32),
    ],
)
def _sc_slot_scan(bstep_hbm, slots_hbm, row_v, out_v):
    wid = lax.axis_index("s") * 2 + lax.axis_index("c")
    pltpu.sync_copy(bstep_hbm.at[wid], row_v)

    iota_f = lax.iota(jnp.int32, LANES).astype(jnp.float32)
    bigf = jnp.float32(1e9)

    def body(i, carry):
        acc_occ, acc_emp = carry
        v = row_v[pl.ds(i * LANES, LANES)]
        gidx_f = iota_f + jnp.float32(i * LANES)
        enc = v.astype(jnp.float32) * jnp.float32(T) + gidx_f
        acc_occ = jnp.minimum(acc_occ, enc)
        acc_emp = jnp.minimum(acc_emp, jnp.where(v == -1, gidx_f, bigf))
        return acc_occ, acc_emp

    acc_occ, acc_emp = lax.fori_loop(
        0, NCHUNK, body,
        (jnp.full((LANES,), 1e9, jnp.float32),
         jnp.full((LANES,), 1e9, jnp.float32)))
    m_occ = acc_occ[0]
    m_emp = acc_emp[0]
    for j in range(1, LANES):
        m_occ = jnp.minimum(m_occ, acc_occ[j])
        m_emp = jnp.minimum(m_emp, acc_emp[j])
    slot_occ = m_occ.astype(jnp.int32) & (T - 1)
    slot = jnp.where(m_emp < bigf, m_emp.astype(jnp.int32), slot_occ)

    out_v[...] = jnp.full((LANES,), 0, jnp.int32) + slot
    pltpu.sync_copy(out_v, slots_hbm.at[wid])


def _scatter_kernel(layer_ref, step_ref, ec_ref, slot_ref, ev_ref, bev_in_ref,
                    bstep_ref, bec_ref, bev_out_ref, bstep_out_ref,
                    bec_out_ref, sem):
    del bev_in_ref
    layer = layer_ref[0]
    step = step_ref[0]
    ec = ec_ref[0]
    slot = slot_ref[layer]

    row_iota = jax.lax.broadcasted_iota(jnp.int32, (L, T), 0)
    col_iota = jax.lax.broadcasted_iota(jnp.int32, (L, T), 1)
    hit = (row_iota == layer) & (col_iota == slot)
    bstep_out_ref[...] = jnp.where(hit, step, bstep_ref[...])
    bec_out_ref[...] = jnp.where(hit, ec, bec_ref[...])

    copy = pltpu.make_async_copy(ev_ref.at[0], bev_out_ref.at[layer, slot], sem)
    copy.start()
    copy.wait()


def kernel(layer, step, evidence, event_count, bank_evidence, bank_step,
           bank_event_count):
    layer_s = jnp.asarray(layer, jnp.int32).reshape(1)
    step_s = jnp.asarray(step, bank_step.dtype).reshape(1)
    ec_s = jnp.asarray(event_count, bank_event_count.dtype).reshape(1)
    ev2 = evidence.astype(bank_evidence.dtype).reshape(1, D)

    slots = _sc_slot_scan(bank_step)

    return pl.pallas_call(
        _scatter_kernel,
        out_shape=(
            jax.ShapeDtypeStruct(bank_evidence.shape, bank_evidence.dtype),
            jax.ShapeDtypeStruct(bank_step.shape, bank_step.dtype),
            jax.ShapeDtypeStruct(bank_event_count.shape, bank_event_count.dtype),
        ),
        in_specs=[
            pl.BlockSpec(memory_space=pltpu.MemorySpace.SMEM),
            pl.BlockSpec(memory_space=pltpu.MemorySpace.SMEM),
            pl.BlockSpec(memory_space=pltpu.MemorySpace.SMEM),
            pl.BlockSpec(memory_space=pltpu.MemorySpace.SMEM),
            pl.BlockSpec(memory_space=pltpu.MemorySpace.VMEM),
            pl.BlockSpec(memory_space=pltpu.MemorySpace.HBM),
            pl.BlockSpec(memory_space=pltpu.MemorySpace.VMEM),
            pl.BlockSpec(memory_space=pltpu.MemorySpace.VMEM),
        ],
        out_specs=(
            pl.BlockSpec(memory_space=pltpu.MemorySpace.HBM),
            pl.BlockSpec(memory_space=pltpu.MemorySpace.VMEM),
            pl.BlockSpec(memory_space=pltpu.MemorySpace.VMEM),
        ),
        input_output_aliases={5: 0},
        scratch_shapes=[pltpu.SemaphoreType.DMA],
    )(layer_s, step_s, ec_s, slots[:, 0], ev2, bank_evidence, bank_step,
      bank_event_count)


# speedup vs baseline: 1.0188x; 1.0111x over previous
import functools

import jax
import jax.numpy as jnp
from jax import lax
from jax.experimental import pallas as pl
from jax.experimental.pallas import tpu as pltpu
from jax.experimental.pallas import tpu_sc as plsc

L, T, D = 32, 1024, 1024
LANES = 16
NCHUNK = T // LANES


@functools.partial(
    pl.kernel,
    out_type=jax.ShapeDtypeStruct((L, LANES), jnp.int32),
    mesh=plsc.VectorSubcoreMesh(core_axis_name="c", subcore_axis_name="s",
                                num_cores=2, num_subcores=16),
    scratch_types=[
        pltpu.VMEM((T,), jnp.int32),
        pltpu.VMEM((LANES,), jnp.int32),
    ],
)
def _sc_slot_scan(bstep_hbm, slots_hbm, row_v, out_v):
    wid = lax.axis_index("s") * 2 + lax.axis_index("c")
    pltpu.sync_copy(bstep_hbm.at[wid], row_v)

    iota_f = lax.iota(jnp.int32, LANES).astype(jnp.float32)
    bigf = jnp.float32(1e9)

    def body(i, carry):
        acc_occ, acc_emp = carry
        v = row_v[pl.ds(i * LANES, LANES)]
        gidx_f = iota_f + jnp.float32(i * LANES)
        enc = v.astype(jnp.float32) * jnp.float32(T) + gidx_f
        acc_occ = jnp.minimum(acc_occ, enc)
        acc_emp = jnp.minimum(acc_emp, jnp.where(v == -1, gidx_f, bigf))
        return acc_occ, acc_emp

    acc_occ, acc_emp = lax.fori_loop(
        0, NCHUNK, body,
        (jnp.full((LANES,), 1e9, jnp.float32),
         jnp.full((LANES,), 1e9, jnp.float32)))
    m_occ = acc_occ[0]
    m_emp = acc_emp[0]
    for j in range(1, LANES):
        m_occ = jnp.minimum(m_occ, acc_occ[j])
        m_emp = jnp.minimum(m_emp, acc_emp[j])
    slot_occ = m_occ.astype(jnp.int32) & (T - 1)
    slot = jnp.where(m_emp < bigf, m_emp.astype(jnp.int32), slot_occ)

    out_v[...] = jnp.full((LANES,), 0, jnp.int32) + slot
    pltpu.sync_copy(out_v, slots_hbm.at[wid])


def _scatter_kernel(layer_ref, step_ref, ec_ref, slot_ref, ev_ref, bev_in_ref,
                    bstep_ref, bec_ref, bev_out_ref, bstep_out_ref,
                    bec_out_ref, sem):
    del bev_in_ref
    layer = layer_ref[0]
    step = step_ref[0]
    ec = ec_ref[0]
    slot = slot_ref[layer, 0]

    row_iota = jax.lax.broadcasted_iota(jnp.int32, (L, T), 0)
    col_iota = jax.lax.broadcasted_iota(jnp.int32, (L, T), 1)
    hit = (row_iota == layer) & (col_iota == slot)
    bstep_out_ref[...] = jnp.where(hit, step, bstep_ref[...])
    bec_out_ref[...] = jnp.where(hit, ec, bec_ref[...])

    copy = pltpu.make_async_copy(ev_ref.at[0], bev_out_ref.at[layer, slot], sem)
    copy.start()
    copy.wait()


def kernel(layer, step, evidence, event_count, bank_evidence, bank_step,
           bank_event_count):
    layer_s = jnp.asarray(layer, jnp.int32).reshape(1)
    step_s = jnp.asarray(step, bank_step.dtype).reshape(1)
    ec_s = jnp.asarray(event_count, bank_event_count.dtype).reshape(1)
    ev2 = evidence.astype(bank_evidence.dtype).reshape(1, D)

    slots = _sc_slot_scan(bank_step)
    bank_evidence, slots = jax.lax.optimization_barrier((bank_evidence, slots))

    return pl.pallas_call(
        _scatter_kernel,
        out_shape=(
            jax.ShapeDtypeStruct(bank_evidence.shape, bank_evidence.dtype),
            jax.ShapeDtypeStruct(bank_step.shape, bank_step.dtype),
            jax.ShapeDtypeStruct(bank_event_count.shape, bank_event_count.dtype),
        ),
        in_specs=[
            pl.BlockSpec(memory_space=pltpu.MemorySpace.SMEM),
            pl.BlockSpec(memory_space=pltpu.MemorySpace.SMEM),
            pl.BlockSpec(memory_space=pltpu.MemorySpace.SMEM),
            pl.BlockSpec(memory_space=pltpu.MemorySpace.SMEM),
            pl.BlockSpec(memory_space=pltpu.MemorySpace.VMEM),
            pl.BlockSpec(memory_space=pltpu.MemorySpace.HBM),
            pl.BlockSpec(memory_space=pltpu.MemorySpace.VMEM),
            pl.BlockSpec(memory_space=pltpu.MemorySpace.VMEM),
        ],
        out_specs=(
            pl.BlockSpec(memory_space=pltpu.MemorySpace.HBM),
            pl.BlockSpec(memory_space=pltpu.MemorySpace.VMEM),
            pl.BlockSpec(memory_space=pltpu.MemorySpace.VMEM),
        ),
        input_output_aliases={5: 0},
        scratch_shapes=[pltpu.SemaphoreType.DMA],
    )(layer_s, step_s, ec_s, slots, ev2, bank_evidence, bank_step,
      bank_event_count)
